# baseline (speedup 1.0000x reference)
import functools

import jax
import jax.numpy as jnp
from jax import lax
from jax.experimental import pallas as pl
from jax.experimental.pallas import tpu as pltpu
from jax.experimental.pallas import tpu_sc as plsc

B = 16384
F = 100
D = 32
DB = D // 8
NCAT = 10000

NC, NS, L = 2, 16, 16
NW = NC * NS
BW = B // NW
NBB = BW // 128
NG = BW // L
STREAMS = 4
SLEN = BW // STREAMS


def _tokenizer_gather(xt, table):
    mesh = plsc.VectorSubcoreMesh(core_axis_name="c", subcore_axis_name="s")

    @functools.partial(
        pl.kernel,
        out_type=jax.ShapeDtypeStruct((F, DB, B // 128, 8, 128), jnp.float32),
        mesh=mesh,
        scratch_types=[
            pltpu.VMEM((2, BW), jnp.int32),
            pltpu.VMEM((2, BW), jnp.int32),
            pltpu.VMEM((2, BW, D), jnp.float32),
            pltpu.VMEM((2, DB, NBB, 8, 128), jnp.float32),
            pltpu.SemaphoreType.DMA,
            pltpu.SemaphoreType.DMA,
            pltpu.SemaphoreType.DMA,
            pltpu.SemaphoreType.DMA,
            pltpu.SemaphoreType.DMA,
            pltpu.SemaphoreType.DMA,
        ],
        compiler_params=pltpu.CompilerParams(use_tc_tiling_on_sc=False,
                                             needs_layout_passes=False,
                                             disable_bounds_checks=True),
    )
    def k(xt_hbm, table_hbm, out_hbm, xv, idxv, rows_v, slab_v,
          si0, si1, sg0, sg1, so0, so1):
        wid = lax.axis_index("s") * NC + lax.axis_index("c")
        b0 = wid * BW
        bb0 = wid * NBB
        iota = lax.iota(jnp.int32, L)
        si = (si0, si1)
        sg = (sg0, sg1)
        so = (so0, so1)

        def in_cp(f, p):
            return pltpu.make_async_copy(
                xt_hbm.at[f, pl.ds(b0, BW)], xv.at[p], si[p])

        def gat_cps(p):
            return [
                pltpu.make_async_copy(
                    table_hbm.at[idxv.at[p, pl.ds(s * SLEN, SLEN)]],
                    rows_v.at[p, pl.ds(s * SLEN, SLEN)], sg[p])
                for s in range(STREAMS)
            ]

        def out_cps(f, p):
            return [
                pltpu.make_async_copy(
                    slab_v.at[p, db],
                    out_hbm.at[f, db, pl.ds(bb0, NBB)], so[p])
                for db in range(DB)
            ]

        zero16 = jnp.zeros((L,), jnp.int32)
        iota32 = iota * D

        def transpose(p):
            def tg_body(tg, _):
                bidx32 = iota32 + tg * (L * D)
                bbq = tg // 8
                k16 = (tg % 8) * L
                for d in range(D):
                    val = plsc.load_gather(rows_v.at[p], [zero16, bidx32 + d])
                    slab_v[p, d // 8, bbq, d % 8, pl.ds(k16, L)] = val
                return 0

            lax.fori_loop(0, NG, tg_body, 0)

        def body(f, p):
            nb = 1 - p
            in_cp(f, p).wait()

            @pl.when(f + 1 < F)
            def _():
                in_cp(f + 1, nb).start()

            off = f * NCAT
            for g in range(NG):
                sl = pl.ds(g * L, L)
                idxv[p, sl] = xv[p, sl] + off

            @pl.when(f >= 3)
            def _():
                for cp in out_cps(f - 3, nb):
                    cp.wait()

            @pl.when(f >= 1)
            def _():
                for cp in gat_cps(nb):
                    cp.wait()

            for cp in gat_cps(p):
                cp.start()

            @pl.when(f >= 1)
            def _():
                transpose(nb)
                for cp in out_cps(f - 1, nb):
                    cp.start()

        in_cp(0, 0).start()

        def pair(i, _):
            body(2 * i, 0)
            body(2 * i + 1, 1)
            return 0

        lax.fori_loop(0, F // 2, pair, 0)

        for cp in gat_cps(1):
            cp.wait()
        for cp in out_cps(F - 3, 1):
            cp.wait()
        transpose(1)
        for cp in out_cps(F - 1, 1):
            cp.start()
        for cp in out_cps(F - 2, 0):
            cp.wait()
        for cp in out_cps(F - 1, 1):
            cp.wait()

    return k(xt, table)


@jax.jit
def kernel(x, table, bias):
    del bias
    out5 = _tokenizer_gather(x.T, table)
    out6 = out5.transpose(2, 4, 0, 1, 3)
    return out6.reshape(B, F, D)


# speedup vs baseline: 1.2945x; 1.2945x over previous
import functools

import jax
import jax.numpy as jnp
from jax import lax
from jax.experimental import pallas as pl
from jax.experimental.pallas import tpu as pltpu
from jax.experimental.pallas import tpu_sc as plsc

B = 16384
F = 100
D = 32
DB = D // 8
NCAT = 10000

NC, NS, L = 2, 16, 16
NW = NC * NS
BW = B // NW
NBB = BW // 128
NG = BW // L
STREAMS = 4
SLEN = BW // STREAMS


def _tokenizer_gather(xt, table):
    mesh = plsc.VectorSubcoreMesh(core_axis_name="c", subcore_axis_name="s")

    @functools.partial(
        pl.kernel,
        out_type=jax.ShapeDtypeStruct((F, DB, B // 128, 8, 128), jnp.float32),
        mesh=mesh,
        scratch_types=[
            pltpu.VMEM((2, BW), jnp.int32),
            pltpu.VMEM((2, BW), jnp.int32),
            pltpu.VMEM((2, BW, D), jnp.float32),
            pltpu.VMEM((2, DB, NBB, 8, 128), jnp.float32),
            pltpu.SemaphoreType.DMA,
            pltpu.SemaphoreType.DMA,
            pltpu.SemaphoreType.DMA,
            pltpu.SemaphoreType.DMA,
            pltpu.SemaphoreType.DMA,
            pltpu.SemaphoreType.DMA,
        ],
        compiler_params=pltpu.CompilerParams(use_tc_tiling_on_sc=False,
                                             needs_layout_passes=False,
                                             disable_bounds_checks=True),
    )
    def k(xt_hbm, table_hbm, out_hbm, xv, idxv, rows_v, slab_v,
          si0, si1, sg0, sg1, so0, so1):
        wid = lax.axis_index("s") * NC + lax.axis_index("c")
        b0 = wid * BW
        bb0 = wid * NBB
        iota = lax.iota(jnp.int32, L)
        si = (si0, si1)
        sg = (sg0, sg1)
        so = (so0, so1)

        def in_cp(f, p):
            return pltpu.make_async_copy(
                xt_hbm.at[f, pl.ds(b0, BW)], xv.at[p], si[p])

        def gat_cps(p):
            return [
                pltpu.make_async_copy(
                    table_hbm.at[idxv.at[p, pl.ds(s * SLEN, SLEN)]],
                    rows_v.at[p, pl.ds(s * SLEN, SLEN)], sg[p])
                for s in range(STREAMS)
            ]

        def out_cps(f, p):
            return [
                pltpu.make_async_copy(
                    slab_v.at[p, db],
                    out_hbm.at[f, db, pl.ds(bb0, NBB)], so[p])
                for db in range(DB)
            ]

        zero16 = jnp.zeros((L,), jnp.int32)
        iota32 = iota * D

        def transpose(p):
            @plsc.parallel_loop(0, NG, unroll=2)
            def tg_body(tg):
                bidx32 = iota32 + tg * (L * D)
                vals = [
                    plsc.load_gather(rows_v.at[p], [zero16, bidx32 + d])
                    for d in range(D)
                ]
                bbq = tg // 8
                k16 = (tg % 8) * L
                for d in range(D):
                    slab_v[p, d // 8, bbq, d % 8, pl.ds(k16, L)] = vals[d]

        def body(f, p):
            nb = 1 - p
            in_cp(f, p).wait()

            @pl.when(f + 1 < F)
            def _():
                in_cp(f + 1, nb).start()

            off = f * NCAT
            for g in range(NG):
                sl = pl.ds(g * L, L)
                idxv[p, sl] = xv[p, sl] + off

            @pl.when(f >= 3)
            def _():
                for cp in out_cps(f - 3, nb):
                    cp.wait()

            @pl.when(f >= 1)
            def _():
                for cp in gat_cps(nb):
                    cp.wait()

            for cp in gat_cps(p):
                cp.start()

            @pl.when(f >= 1)
            def _():
                transpose(nb)
                for cp in out_cps(f - 1, nb):
                    cp.start()

        in_cp(0, 0).start()

        def pair(i, _):
            body(2 * i, 0)
            body(2 * i + 1, 1)
            return 0

        lax.fori_loop(0, F // 2, pair, 0)

        for cp in gat_cps(1):
            cp.wait()
        for cp in out_cps(F - 3, 1):
            cp.wait()
        transpose(1)
        for cp in out_cps(F - 1, 1):
            cp.start()
        for cp in out_cps(F - 2, 0):
            cp.wait()
        for cp in out_cps(F - 1, 1):
            cp.wait()

    return k(xt, table)


@jax.jit
def kernel(x, table, bias):
    del bias
    out5 = _tokenizer_gather(x.T, table)
    out6 = out5.transpose(2, 4, 0, 1, 3)
    return out6.reshape(B, F, D)


# speedup vs baseline: 1.3066x; 1.0093x over previous
import functools

import jax
import jax.numpy as jnp
from jax import lax
from jax.experimental import pallas as pl
from jax.experimental.pallas import tpu as pltpu
from jax.experimental.pallas import tpu_sc as plsc

B = 16384
F = 100
D = 32
DB = D // 8
NCAT = 10000
SLAB_PAD = 128

NC, NS, L = 2, 16, 16
NW = NC * NS
BW = B // NW
NBB = BW // 128
NG = BW // L
STREAMS = 4
SLEN = BW // STREAMS


def _tokenizer_gather(xt, table):
    mesh = plsc.VectorSubcoreMesh(core_axis_name="c", subcore_axis_name="s")

    @functools.partial(
        pl.kernel,
        out_type=jax.ShapeDtypeStruct((F, DB, B // 128, 8, 128), jnp.float32),
        mesh=mesh,
        scratch_types=[
            pltpu.VMEM((2, BW), jnp.int32),
            pltpu.VMEM((2, BW), jnp.int32),
            pltpu.VMEM((2, BW, D), jnp.float32),
            pltpu.VMEM((2, DB, NBB, 8, SLAB_PAD), jnp.float32),
            pltpu.SemaphoreType.DMA,
            pltpu.SemaphoreType.DMA,
            pltpu.SemaphoreType.DMA,
            pltpu.SemaphoreType.DMA,
            pltpu.SemaphoreType.DMA,
            pltpu.SemaphoreType.DMA,
        ],
        compiler_params=pltpu.CompilerParams(use_tc_tiling_on_sc=False,
                                             needs_layout_passes=False,
                                             disable_bounds_checks=True),
    )
    def k(xt_hbm, table_hbm, out_hbm, xv, idxv, rows_v, slab_v,
          si0, si1, sg0, sg1, so0, so1):
        wid = lax.axis_index("s") * NC + lax.axis_index("c")
        b0 = wid * BW
        bb0 = wid * NBB
        iota = lax.iota(jnp.int32, L)
        si = (si0, si1)
        sg = (sg0, sg1)
        so = (so0, so1)

        def in_cp(f, p):
            return pltpu.make_async_copy(
                xt_hbm.at[f, pl.ds(b0, BW)], xv.at[p], si[p])

        def gat_cps(p):
            return [
                pltpu.make_async_copy(
                    table_hbm.at[idxv.at[p, pl.ds(s * SLEN, SLEN)]],
                    rows_v.at[p, pl.ds(s * SLEN, SLEN)], sg[p])
                for s in range(STREAMS)
            ]

        def out_cps(f, p):
            return [
                pltpu.make_async_copy(
                    slab_v.at[p, db, pl.ds(0, NBB), pl.ds(0, 8), pl.ds(0, 128)],
                    out_hbm.at[f, db, pl.ds(bb0, NBB)], so[p])
                for db in range(DB)
            ]

        zero16 = jnp.zeros((L,), jnp.int32)
        SLAB_D8 = SLAB_PAD
        SLAB_DB = NBB * 8 * SLAB_D8
        doff_low = (iota >> 3) * SLAB_DB + (iota & 7) * SLAB_D8

        def transpose(p):
            @plsc.parallel_loop(0, BW, unroll=4)
            def b_body(b):
                base = (b >> 7) * (8 * SLAB_D8) + (b & 127)
                dlow = doff_low + base
                dhigh = dlow + 2 * SLAB_DB
                v0 = rows_v[p, b, pl.ds(0, L)]
                v1 = rows_v[p, b, pl.ds(L, L)]
                plsc.store_scatter(slab_v.at[p],
                                   [zero16, zero16, zero16, dlow], v0)
                plsc.store_scatter(slab_v.at[p],
                                   [zero16, zero16, zero16, dhigh], v1)

        def body(f, p):
            nb = 1 - p
            in_cp(f, p).wait()

            @pl.when(f + 1 < F)
            def _():
                in_cp(f + 1, nb).start()

            off = f * NCAT
            for g in range(NG):
                sl = pl.ds(g * L, L)
                idxv[p, sl] = xv[p, sl] + off

            @pl.when(f >= 3)
            def _():
                for cp in out_cps(f - 3, nb):
                    cp.wait()

            @pl.when(f >= 1)
            def _():
                for cp in gat_cps(nb):
                    cp.wait()

            for cp in gat_cps(p):
                cp.start()

            @pl.when(f >= 1)
            def _():
                transpose(nb)
                for cp in out_cps(f - 1, nb):
                    cp.start()

        in_cp(0, 0).start()

        def pair(i, _):
            body(2 * i, 0)
            body(2 * i + 1, 1)
            return 0

        lax.fori_loop(0, F // 2, pair, 0)

        for cp in gat_cps(1):
            cp.wait()
        for cp in out_cps(F - 3, 1):
            cp.wait()
        transpose(1)
        for cp in out_cps(F - 1, 1):
            cp.start()
        for cp in out_cps(F - 2, 0):
            cp.wait()
        for cp in out_cps(F - 1, 1):
            cp.wait()

    return k(xt, table)


@jax.jit
def kernel(x, table, bias):
    del bias
    out5 = _tokenizer_gather(x.T, table)
    out6 = out5.transpose(2, 4, 0, 1, 3)
    return out6.reshape(B, F, D)


# speedup vs baseline: 2.5587x; 1.9582x over previous
import functools

import jax
import jax.numpy as jnp
from jax import lax
from jax.experimental import pallas as pl
from jax.experimental.pallas import tpu as pltpu
from jax.experimental.pallas import tpu_sc as plsc

B = 16384
F = 100
D = 32
DB = D // 8
NCAT = 10000
SLAB_PAD = 136

NC, NS, L = 2, 16, 16
NW = NC * NS
BW = B // NW
NBB = BW // 128
NG = BW // L
STREAMS = 4
SLEN = BW // STREAMS


def _tokenizer_gather(xt, table):
    mesh = plsc.VectorSubcoreMesh(core_axis_name="c", subcore_axis_name="s")

    @functools.partial(
        pl.kernel,
        out_type=jax.ShapeDtypeStruct((F, DB, B // 128, 8, 128), jnp.float32),
        mesh=mesh,
        scratch_types=[
            pltpu.VMEM((2, BW), jnp.int32),
            pltpu.VMEM((2, BW), jnp.int32),
            pltpu.VMEM((2, BW, D), jnp.float32),
            pltpu.VMEM((2, DB, NBB, 8, SLAB_PAD), jnp.float32),
            pltpu.SemaphoreType.DMA,
            pltpu.SemaphoreType.DMA,
            pltpu.SemaphoreType.DMA,
            pltpu.SemaphoreType.DMA,
            pltpu.SemaphoreType.DMA,
            pltpu.SemaphoreType.DMA,
        ],
        compiler_params=pltpu.CompilerParams(use_tc_tiling_on_sc=False,
                                             needs_layout_passes=False,
                                             disable_bounds_checks=True),
    )
    def k(xt_hbm, table_hbm, out_hbm, xv, idxv, rows_v, slab_v,
          si0, si1, sg0, sg1, so0, so1):
        wid = lax.axis_index("s") * NC + lax.axis_index("c")
        b0 = wid * BW
        bb0 = wid * NBB
        iota = lax.iota(jnp.int32, L)
        si = (si0, si1)
        sg = (sg0, sg1)
        so = (so0, so1)

        def in_cp(f, p):
            return pltpu.make_async_copy(
                xt_hbm.at[f, pl.ds(b0, BW)], xv.at[p], si[p])

        def gat_cps(p):
            return [
                pltpu.make_async_copy(
                    table_hbm.at[idxv.at[p, pl.ds(s * SLEN, SLEN)]],
                    rows_v.at[p, pl.ds(s * SLEN, SLEN)], sg[p])
                for s in range(STREAMS)
            ]

        def out_cps(f, p):
            return [
                pltpu.make_async_copy(
                    slab_v.at[p, db, pl.ds(0, NBB), pl.ds(0, 8), pl.ds(0, 128)],
                    out_hbm.at[f, db, pl.ds(bb0, NBB)], so[p])
                for db in range(DB)
            ]

        zero16 = jnp.zeros((L,), jnp.int32)
        SLAB_D8 = SLAB_PAD
        SLAB_DB = NBB * 8 * SLAB_D8
        doff_low = (iota >> 3) * SLAB_DB + (iota & 7) * SLAB_D8

        def transpose(p):
            @plsc.parallel_loop(0, BW, unroll=4)
            def b_body(b):
                base = (b >> 7) * (8 * SLAB_D8) + (b & 127)
                dlow = doff_low + base
                dhigh = dlow + 2 * SLAB_DB
                v0 = rows_v[p, b, pl.ds(0, L)]
                v1 = rows_v[p, b, pl.ds(L, L)]
                plsc.store_scatter(slab_v.at[p],
                                   [zero16, zero16, zero16, dlow], v0)
                plsc.store_scatter(slab_v.at[p],
                                   [zero16, zero16, zero16, dhigh], v1)

        def body(f, p):
            nb = 1 - p
            in_cp(f, p).wait()

            @pl.when(f + 1 < F)
            def _():
                in_cp(f + 1, nb).start()

            off = f * NCAT
            for g in range(NG):
                sl = pl.ds(g * L, L)
                idxv[p, sl] = xv[p, sl] + off

            @pl.when(f >= 3)
            def _():
                for cp in out_cps(f - 3, nb):
                    cp.wait()

            @pl.when(f >= 1)
            def _():
                for cp in gat_cps(nb):
                    cp.wait()

            for cp in gat_cps(p):
                cp.start()

            @pl.when(f >= 1)
            def _():
                transpose(nb)
                for cp in out_cps(f - 1, nb):
                    cp.start()

        in_cp(0, 0).start()

        def pair(i, _):
            body(2 * i, 0)
            body(2 * i + 1, 1)
            return 0

        lax.fori_loop(0, F // 2, pair, 0)

        for cp in gat_cps(1):
            cp.wait()
        for cp in out_cps(F - 3, 1):
            cp.wait()
        transpose(1)
        for cp in out_cps(F - 1, 1):
            cp.start()
        for cp in out_cps(F - 2, 0):
            cp.wait()
        for cp in out_cps(F - 1, 1):
            cp.wait()

    return k(xt, table)


@jax.jit
def kernel(x, table, bias):
    del bias
    out5 = _tokenizer_gather(x.T, table)
    out6 = out5.transpose(2, 4, 0, 1, 3)
    return out6.reshape(B, F, D)
